# baseline (speedup 1.0000x reference)
import functools

import jax
import jax.numpy as jnp
from jax import lax
from jax.experimental import pallas as pl
from jax.experimental.pallas import tpu as pltpu
from jax.experimental.pallas import tpu_sc as plsc

_VOCAB = 1000000
_DIM = 64
_PAD = 128
_P = 0.1
_NC = 2
_NS = 16
_NW = _NC * _NS
_LANES = 16
_CHUNK = 256
_S = 512000
_BV = 16000
_BB = 256


def _transpose_pack(wt):
    def body(x1_ref, x2_ref, out_ref):
        out_ref[...] = jnp.concatenate([x1_ref[...].T, x2_ref[...].T], axis=1)

    nb = _S // _BV
    last = _VOCAB // _BV
    return pl.pallas_call(
        body,
        grid=(nb,),
        in_specs=[pl.BlockSpec((_DIM, _BV), lambda i: (0, i)),
                  pl.BlockSpec((_DIM, _BV),
                               lambda i: (0, jnp.minimum(i + nb, last)))],
        out_specs=pl.BlockSpec((_BV, _PAD), lambda i: (i, 0)),
        out_shape=jax.ShapeDtypeStruct((_S, _PAD), jnp.float32),
    )(wt, wt)


def _format_out(padded, b, l):
    def body(x_ref, o_ref):
        xb = x_ref[...][:, :_DIM].reshape(_BB, l, _DIM)
        o_ref[...] = jnp.transpose(xb, (1, 2, 0))

    return pl.pallas_call(
        body,
        grid=(b // _BB,),
        in_specs=[pl.BlockSpec((_BB * l, _PAD), lambda i: (i, 0))],
        out_specs=pl.BlockSpec((l, _DIM, _BB), lambda i: (0, 0, i)),
        out_shape=jax.ShapeDtypeStruct((l, _DIM, b), jnp.float32),
    )(padded)


def _dropout_gather(ridx, scale2, table):
    n = ridx.shape[0]
    per_w = n // _NW
    assert per_w * _NW == n and per_w % _CHUNK == 0
    nchunks = per_w // _CHUNK
    mesh = plsc.VectorSubcoreMesh(core_axis_name="c", subcore_axis_name="s")

    @functools.partial(
        pl.kernel,
        out_type=jax.ShapeDtypeStruct((n, _PAD), jnp.float32),
        mesh=mesh,
        scratch_types=[
            pltpu.VMEM((per_w,), jnp.int32),
            pltpu.VMEM((per_w,), jnp.float32),
            pltpu.VMEM((3, _CHUNK, _DIM), jnp.float32),
            pltpu.VMEM((2, _CHUNK, _PAD), jnp.float32),
            pltpu.SemaphoreType.DMA,
            pltpu.SemaphoreType.DMA,
        ],
        compiler_params=pltpu.CompilerParams(use_tc_tiling_on_sc=False),
    )
    def k(idx_hbm, scale_hbm, table_hbm, out_hbm,
          idx_v, s_v, rows_v, obuf_v, gsem, osem):
        wid = lax.axis_index("s") * _NC + lax.axis_index("c")
        base = wid * per_w
        pltpu.sync_copy(idx_hbm.at[pl.ds(base, per_w)], idx_v)
        pltpu.async_copy(scale_hbm.at[idx_v], s_v, gsem).wait()

        def gather_chunk(c):
            return pltpu.async_copy(
                table_hbm.at[idx_v.at[pl.ds(c * _CHUNK, _CHUNK)]],
                rows_v.at[c % 3], gsem)

        gcopies = {0: gather_chunk(0), 1: gather_chunk(1)}
        ocopies = {}
        for c in range(nchunks):
            buf = c % 3
            ob = c % 2
            gcopies.pop(c).wait()
            if c - 2 in ocopies:
                ocopies.pop(c - 2).wait()

            def mul_rows16(j, carry, c=c, buf=buf, ob=ob):
                svec = s_v[pl.ds(c * _CHUNK + j * _LANES, _LANES)]
                for r in range(_LANES):
                    s = svec.at[jnp.full((_LANES,), r, jnp.int32)].get(
                        mode="promise_in_bounds")
                    row = j * _LANES + r
                    for g in range(_DIM // _LANES):
                        sl = pl.ds(g * _LANES, _LANES)
                        obuf_v[ob, row, sl] = rows_v[buf, row, sl] * s
                return carry

            lax.fori_loop(0, _CHUNK // _LANES, mul_rows16, 0)
            ocopies[c] = pltpu.async_copy(
                obuf_v.at[ob],
                out_hbm.at[pl.ds(base + c * _CHUNK, _CHUNK)],
                osem)
            if c + 2 < nchunks:
                gcopies[c + 2] = gather_chunk(c + 2)
        for c in sorted(ocopies):
            ocopies.pop(c).wait()

    return k(ridx, scale2, table)


def kernel(words, weight):
    b, l = words.shape
    keep = jax.random.bernoulli(jax.random.key(42), 1.0 - _P, (_VOCAB,))
    scale = keep.astype(weight.dtype) / (1.0 - _P)
    shi = jnp.concatenate(
        [scale[_S:], jnp.zeros(2 * _S - _VOCAB, jnp.float32)])
    scale2 = jnp.stack([scale[:_S], shi], axis=1).reshape(2 * _S)
    table = _transpose_pack(weight.T).reshape(2 * _S, _DIM)
    idx = words.reshape(b * l)
    ridx = jnp.where(idx < _S, 2 * idx, 2 * (idx - _S) + 1)
    padded = _dropout_gather(ridx, scale2, table)
    out = _format_out(padded, b, l)
    return jnp.transpose(out, (2, 0, 1))


# speedup vs baseline: 1.7110x; 1.7110x over previous
import functools

import jax
import jax.numpy as jnp
from jax import lax
from jax.experimental import pallas as pl
from jax.experimental.pallas import tpu as pltpu
from jax.experimental.pallas import tpu_sc as plsc

_VOCAB = 1000000
_DIM = 64
_PAD = 128
_P = 0.1
_NC = 2
_NS = 16
_NW = _NC * _NS
_LANES = 16
_CHUNK = 256
_BV = 32768
_BB = 256


def _transpose_pad(wt):
    def body(wt_ref, out_ref):
        x = wt_ref[...]
        out_ref[...] = jnp.concatenate(
            [x.T, jnp.zeros((x.shape[1], _PAD - _DIM), jnp.float32)], axis=1
        )

    return pl.pallas_call(
        body,
        grid=(pl.cdiv(_VOCAB, _BV),),
        in_specs=[pl.BlockSpec((_DIM, _BV), lambda i: (0, i))],
        out_specs=pl.BlockSpec((_BV, _PAD), lambda i: (i, 0)),
        out_shape=jax.ShapeDtypeStruct((_VOCAB, _PAD), jnp.float32),
    )(wt)


def _format_out(padded, b, l):
    def body(x_ref, o_ref):
        xb = x_ref[...][:, :_DIM].reshape(_BB, l, _DIM)
        o_ref[...] = jnp.transpose(xb, (1, 2, 0))

    return pl.pallas_call(
        body,
        grid=(b // _BB,),
        in_specs=[pl.BlockSpec((_BB * l, _PAD), lambda i: (i, 0))],
        out_specs=pl.BlockSpec((l, _DIM, _BB), lambda i: (0, 0, i)),
        out_shape=jax.ShapeDtypeStruct((l, _DIM, b), jnp.float32),
    )(padded)


def _dropout_gather(idx, scale, table):
    n = idx.shape[0]
    per_w = n // _NW
    assert per_w * _NW == n and per_w % _CHUNK == 0
    nchunks = per_w // _CHUNK
    mesh = plsc.VectorSubcoreMesh(core_axis_name="c", subcore_axis_name="s")

    @functools.partial(
        pl.kernel,
        out_type=jax.ShapeDtypeStruct((n, _PAD), jnp.float32),
        mesh=mesh,
        scratch_types=[
            pltpu.VMEM((per_w,), jnp.int32),
            pltpu.VMEM((per_w,), jnp.float32),
            pltpu.VMEM((3, _CHUNK, _PAD), jnp.float32),
            pltpu.SemaphoreType.DMA,
            pltpu.SemaphoreType.DMA,
        ],
        compiler_params=pltpu.CompilerParams(use_tc_tiling_on_sc=True),
    )
    def k(idx_hbm, scale_hbm, table_hbm, out_hbm,
          idx_v, s_v, rows_v, gsem, osem):
        wid = lax.axis_index("s") * _NC + lax.axis_index("c")
        base = wid * per_w
        pltpu.sync_copy(idx_hbm.at[pl.ds(base, per_w)], idx_v)
        pltpu.async_copy(scale_hbm.at[idx_v], s_v, gsem).wait()

        def gather_chunk(c):
            return pltpu.async_copy(
                table_hbm.at[idx_v.at[pl.ds(c * _CHUNK, _CHUNK)]],
                rows_v.at[c % 3], gsem)

        gcopies = {0: gather_chunk(0), 1: gather_chunk(1)}
        ocopies = {}
        for c in range(nchunks):
            buf = c % 3
            gcopies.pop(c).wait()

            def mul_rows16(j, carry, c=c, buf=buf):
                svec = s_v[pl.ds(c * _CHUNK + j * _LANES, _LANES)]
                for r in range(_LANES):
                    s = svec.at[jnp.full((_LANES,), r, jnp.int32)].get(
                        mode="promise_in_bounds")
                    row = j * _LANES + r
                    for g in range(_DIM // _LANES):
                        sl = pl.ds(g * _LANES, _LANES)
                        rows_v[buf, row, sl] = rows_v[buf, row, sl] * s
                return carry

            lax.fori_loop(0, _CHUNK // _LANES, mul_rows16, 0)
            ocopies[c] = pltpu.async_copy(
                rows_v.at[buf],
                out_hbm.at[pl.ds(base + c * _CHUNK, _CHUNK)],
                osem)
            if c + 2 < nchunks:
                if c - 1 in ocopies:
                    ocopies.pop(c - 1).wait()
                gcopies[c + 2] = gather_chunk(c + 2)
        for c in sorted(ocopies):
            ocopies.pop(c).wait()

    return k(idx, scale, table)


def kernel(words, weight):
    b, l = words.shape
    keep = jax.random.bernoulli(jax.random.key(42), 1.0 - _P, (_VOCAB,))
    scale = keep.astype(weight.dtype) / (1.0 - _P)
    table = _transpose_pad(weight.T)
    idx = words.reshape(b * l)
    padded = _dropout_gather(idx, scale, table)
    out = _format_out(padded, b, l)
    return jnp.transpose(out, (2, 0, 1))
